# baseline (speedup 1.0000x reference)
import functools

import jax
import jax.numpy as jnp
from jax.experimental import pallas as pl
from jax.experimental.pallas import tpu as pltpu

_D = 384
_COSTS = (0.05, 0.25, 0.6)


def _first_min_idx(d2, n_codes):
    m = jnp.min(d2, axis=1, keepdims=True)
    eq = (d2 == m).astype(jnp.float32)
    iota_s = jax.lax.broadcasted_iota(jnp.int32, (n_codes, n_codes), 0)
    w = jax.lax.bitcast_convert_type((127 - iota_s) << 23, jnp.float32)
    se = jax.lax.dot_general(eq, w, (((1,), (0,)), ((), ())),
                             preferred_element_type=jnp.float32)
    ebits = jax.lax.shift_right_logical(
        jax.lax.bitcast_convert_type(se, jnp.int32), 23)
    idx_rep = 127 - ebits
    return idx_rep, m


def _distances(x, cb, n_codes):
    cb2 = jnp.sum(cb * cb, axis=1)[None, :]
    xc = jax.lax.dot_general(x, cb, (((1,), (1,)), ((), ())),
                             preferred_element_type=jnp.float32)
    ones = jnp.ones((_D, n_codes), jnp.float32)
    x2 = jax.lax.dot_general(x * x, ones, (((1,), (0,)), ((), ())),
                             preferred_element_type=jnp.float32)
    return x2 - 2.0 * xc + cb2


def _vq_body(x_ref, cb_ref, idx_ref, loss_ref, *, n_codes):
    x = x_ref[...]
    cb = cb_ref[...]
    d2 = _distances(x, cb, n_codes)
    idx_rep, m = _first_min_idx(d2, n_codes)
    idx_ref[...] = idx_rep[:, 0]
    s = jnp.sum(m)

    @pl.when(pl.program_id(0) == 0)
    def _init():
        loss_ref[0, 0] = 0.0

    loss_ref[0, 0] += s


def _vq_body_q(x_ref, cb_ref, idx_ref, loss_ref, q_ref, *, n_codes):
    x = x_ref[...]
    cb = cb_ref[...]
    d2 = _distances(x, cb, n_codes)
    idx_rep, m = _first_min_idx(d2, n_codes)
    idx_ref[...] = idx_rep[:, 0]
    iota = jax.lax.broadcasted_iota(jnp.int32, d2.shape, 1)
    onehot = (iota == idx_rep).astype(jnp.float32)
    cb_hi = cb.astype(jnp.bfloat16).astype(jnp.float32)
    cb_lo = cb - cb_hi
    dn = (((1,), (0,)), ((), ()))
    q_hi = jax.lax.dot_general(onehot, cb_hi, dn,
                               preferred_element_type=jnp.float32)
    q_lo = jax.lax.dot_general(onehot, cb_lo, dn,
                               preferred_element_type=jnp.float32)
    q_ref[...] = q_hi + q_lo
    s = jnp.sum(m)

    @pl.when(pl.program_id(0) == 0)
    def _init():
        loss_ref[0, 0] = 0.0

    loss_ref[0, 0] += s


def _vq_level(x_flat, cb, block_rows, with_q):
    n, d = x_flat.shape
    k = cb.shape[0]
    grid = n // block_rows
    out_specs = [
        pl.BlockSpec((block_rows,), lambda i: (i,)),
        pl.BlockSpec((1, 1), lambda i: (0, 0), memory_space=pltpu.SMEM),
    ]
    out_shape = [
        jax.ShapeDtypeStruct((n,), jnp.int32),
        jax.ShapeDtypeStruct((1, 1), jnp.float32),
    ]
    if with_q:
        body = functools.partial(_vq_body_q, n_codes=k)
        out_specs.append(pl.BlockSpec((block_rows, d), lambda i: (i, 0)))
        out_shape.append(jax.ShapeDtypeStruct((n, d), jnp.float32))
    else:
        body = functools.partial(_vq_body, n_codes=k)
    outs = pl.pallas_call(
        body,
        grid=(grid,),
        in_specs=[
            pl.BlockSpec((block_rows, d), lambda i: (i, 0)),
            pl.BlockSpec((k, d), lambda i: (0, 0)),
        ],
        out_specs=out_specs,
        out_shape=out_shape,
    )(x_flat, cb)
    if with_q:
        idx, loss_sum, q = outs
        return idx, loss_sum[0, 0], q
    idx, loss_sum = outs
    return idx, loss_sum[0, 0], None


def kernel(l0, l1, l2, cb0, cb1, cb2):
    x0 = l0.reshape(-1, _D)
    x1 = l1.reshape(-1, _D)
    x2 = l2.reshape(-1, _D)
    idx0, s0, q0 = _vq_level(x0, cb0, 2048, True)
    idx1, s1, q1 = _vq_level(x1, cb1, 2048, True)
    idx2, s2, q2 = _vq_level(x2, cb2, 4096, True)
    total = (
        (1.0 + _COSTS[0]) * s0 / l0.size
        + (1.0 + _COSTS[1]) * s1 / l1.size
        + (1.0 + _COSTS[2]) * s2 / l2.size
    )
    return (idx0.reshape(l0.shape[:-1]), idx1.reshape(l1.shape[:-1]),
            idx2.reshape(l2.shape[:-1]), total,
            q0.reshape(l0.shape), q1.reshape(l1.shape), q2.reshape(l2.shape))
